# baseline (speedup 1.0000x reference)
import functools

import jax
import jax.numpy as jnp
from jax import lax
from jax.experimental import pallas as pl
from jax.experimental.pallas import tpu as pltpu
from jax.experimental.pallas import tpu_sc as plsc


def _positional_encoding(context_size, embedding_size):
    positions = jnp.arange(context_size, dtype=jnp.float32)
    indices = jnp.arange(embedding_size // 2, dtype=jnp.float32)
    scaling_factor = 10000.0 ** (2.0 * indices / embedding_size)
    angles = positions[:, None] / scaling_factor[None, :]
    pe = jnp.stack([jnp.sin(angles), jnp.cos(angles)], axis=-1)
    return pe.reshape(context_size, embedding_size)


def kernel(x, table):
    batch, context = x.shape
    _, d = table.shape
    n = batch * context

    info = plsc.get_sparse_core_info()
    nw = info.num_cores * info.num_subcores
    bpw = n // nw

    pe = _positional_encoding(context, d)
    mesh = plsc.VectorSubcoreMesh(core_axis_name="c", subcore_axis_name="s")

    @functools.partial(
        pl.kernel,
        out_type=jax.ShapeDtypeStruct((n, d), jnp.float32),
        mesh=mesh,
        scratch_types=[
            pltpu.VMEM((bpw,), jnp.int32),
            pltpu.VMEM((bpw, d), jnp.float32),
            pltpu.VMEM((bpw, d), jnp.float32),
            pltpu.SemaphoreType.DMA,
            pltpu.SemaphoreType.DMA,
        ],
        compiler_params=pltpu.CompilerParams(use_tc_tiling_on_sc=False),
    )
    def emb(x_hbm, pe_hbm, table_hbm, out_hbm, idx_v, rows_v, pe_v, gsem, psem):
        wid = lax.axis_index("s") * info.num_cores + lax.axis_index("c")
        base = wid * bpw
        pe_base = lax.rem(base, context)
        pe_cp = pltpu.make_async_copy(pe_hbm.at[pl.ds(pe_base, bpw)], pe_v, psem)
        pe_cp.start()
        pltpu.sync_copy(x_hbm.at[pl.ds(base, bpw)], idx_v)
        pltpu.async_copy(table_hbm.at[idx_v], rows_v, gsem).wait()
        pe_cp.wait()

        def add_row(i, carry):
            for j in range(d // 16):
                s = pl.ds(j * 16, 16)
                rows_v[i, s] = rows_v[i, s] + pe_v[i, s]
            return carry

        lax.fori_loop(0, bpw, add_row, 0)
        pltpu.sync_copy(rows_v, out_hbm.at[pl.ds(base, bpw)])

    out = emb(x.reshape(n).astype(jnp.int32), pe, table)
    return out.reshape(batch, context, d)


# speedup vs baseline: 2.4773x; 2.4773x over previous
import functools

import jax
import jax.numpy as jnp
from jax import lax
from jax.experimental import pallas as pl
from jax.experimental.pallas import tpu as pltpu
from jax.experimental.pallas import tpu_sc as plsc


def _positional_encoding(context_size, embedding_size):
    positions = jnp.arange(context_size, dtype=jnp.float32)
    indices = jnp.arange(embedding_size // 2, dtype=jnp.float32)
    scaling_factor = 10000.0 ** (2.0 * indices / embedding_size)
    angles = positions[:, None] / scaling_factor[None, :]
    pe = jnp.stack([jnp.sin(angles), jnp.cos(angles)], axis=-1)
    return pe.reshape(context_size, embedding_size)


def kernel(x, table):
    batch, context = x.shape
    v, d = table.shape
    n = batch * context

    info = plsc.get_sparse_core_info()
    nw = info.num_cores * info.num_subcores
    bpw = n // nw
    grp = 32
    ngroups = bpw // grp

    pe = _positional_encoding(context, d)
    table3 = table.reshape(v // 8, 8, d)
    x_flat = x.reshape(n).astype(jnp.int32)

    mesh = plsc.VectorSubcoreMesh(core_axis_name="c", subcore_axis_name="s")

    @functools.partial(
        pl.kernel,
        out_type=jax.ShapeDtypeStruct((n, d), jnp.float32),
        mesh=mesh,
        scratch_types=[
            pltpu.VMEM((bpw,), jnp.int32),
            pltpu.VMEM((bpw, d), jnp.float32),
            pltpu.VMEM((bpw, d), jnp.float32),
            pltpu.SemaphoreType.DMA,
            pltpu.SemaphoreType.DMA,
            pltpu.SemaphoreType.DMA,
        ],
    )
    def emb(x_hbm, pe_hbm, t3_hbm, out_hbm,
            idx_v, rows_v, pe_v, sem_a, sem_b, psem):
        wid = lax.axis_index("s") * info.num_cores + lax.axis_index("c")
        base = wid * bpw
        pe_cp = pltpu.make_async_copy(
            pe_hbm.at[pl.ds(lax.rem(base, context), bpw)], pe_v, psem)
        pe_cp.start()
        pltpu.sync_copy(x_hbm.at[pl.ds(base, bpw)], idx_v)

        sems = [sem_a, sem_b]

        def issue(g):
            cps = []
            for h in range(grp // 16):
                iv = idx_v[pl.ds(g * grp + h * 16, 16)]
                gv = lax.shift_right_logical(iv, 3)
                sv = lax.rem(iv, 8)
                for l in range(16):
                    row = g * grp + h * 16 + l
                    cp = pltpu.make_async_copy(
                        t3_hbm.at[gv[l], sv[l]], rows_v.at[row], sems[g % 2])
                    cp.start()
                    cps.append(cp)
            return cps

        inflight = {0: issue(0)}
        pe_cp.wait()
        for g in range(ngroups):
            if g + 1 < ngroups:
                inflight[g + 1] = issue(g + 1)
            for cp in inflight.pop(g):
                cp.wait()

            def add_body(i, carry):
                row = g * grp + i
                for j in range(d // 16):
                    s = pl.ds(j * 16, 16)
                    rows_v[row, s] = rows_v[row, s] + pe_v[row, s]
                return carry

            lax.fori_loop(0, grp, add_body, 0)

        pltpu.sync_copy(rows_v, out_hbm.at[pl.ds(base, bpw)])

    out = emb(x_flat, pe, table3)
    return out.reshape(batch, context, d)


# speedup vs baseline: 4.4517x; 1.7970x over previous
import functools

import jax
import jax.numpy as jnp
from jax import lax
from jax.experimental import pallas as pl
from jax.experimental.pallas import tpu as pltpu
from jax.experimental.pallas import tpu_sc as plsc


def _positional_encoding(context_size, embedding_size):
    positions = jnp.arange(context_size, dtype=jnp.float32)
    indices = jnp.arange(embedding_size // 2, dtype=jnp.float32)
    scaling_factor = 10000.0 ** (2.0 * indices / embedding_size)
    angles = positions[:, None] / scaling_factor[None, :]
    pe = jnp.stack([jnp.sin(angles), jnp.cos(angles)], axis=-1)
    return pe.reshape(context_size, embedding_size)


def kernel(x, table):
    batch, context = x.shape
    v, d = table.shape
    n = batch * context

    info = plsc.get_sparse_core_info()
    nw = info.num_cores * info.num_subcores
    bpw = n // nw
    wv = 4
    nwaves = bpw // wv
    nt = bpw // 128

    pe_t = _positional_encoding(context, d).T.reshape(d, context // 128, 128)
    t3 = table.T.reshape(d // 8, 8, v)
    x_flat = x.reshape(n).astype(jnp.int32)

    mesh = plsc.VectorSubcoreMesh(core_axis_name="c", subcore_axis_name="s")

    @functools.partial(
        pl.kernel,
        out_type=jax.ShapeDtypeStruct((batch, d, context // 128, 128), jnp.float32),
        mesh=mesh,
        scratch_types=[
            pltpu.VMEM((bpw,), jnp.int32),
            pltpu.SMEM((bpw,), jnp.int32),
            pltpu.VMEM((2, wv, d // 8, 8, 128), jnp.float32),
            pltpu.VMEM((d, nt, 128), jnp.float32),
            pltpu.VMEM((d, nt, 128), jnp.float32),
            pltpu.SemaphoreType.DMA,
            pltpu.SemaphoreType.DMA,
            pltpu.SemaphoreType.DMA,
        ],
        compiler_params=pltpu.CompilerParams(needs_layout_passes=False),
    )
    def emb(x_hbm, pe_hbm, t3_hbm, out_hbm,
            idx_v, ism, gbuf, cols_v, pe_v, sem_a, sem_b, psem):
        wid = lax.axis_index("s") * info.num_cores + lax.axis_index("c")
        base = wid * bpw
        b = lax.div(base, context)
        t0 = pl.multiple_of(lax.rem(base, context), bpw)
        tg0 = lax.div(t0, 128)
        pe_cp = pltpu.make_async_copy(
            pe_hbm.at[:, pl.ds(tg0, nt), :], pe_v, psem)
        pe_cp.start()
        pltpu.sync_copy(x_hbm.at[pl.ds(base, bpw)], idx_v)

        def stage(q, carry):
            iv = idx_v[pl.ds(q * 16, 16)]
            for l in range(16):
                ism[q * 16 + l] = iv[l]
            return carry

        lax.fori_loop(0, bpw // 16, stage, 0)

        sems = [sem_a, sem_b]
        iota = lax.iota(jnp.int32, 16)
        chunk_hi = [(j * 16 + iota) >> 3 for j in range(d // 16)]
        chunk_lo = [(j * 16 + iota) & 7 for j in range(d // 16)]

        def issue(w, parity):
            for l in range(wv):
                r = ism[w * wv + l]
                r128 = pl.multiple_of((r >> 7) * 128, 128)
                pltpu.make_async_copy(
                    t3_hbm.at[:, :, pl.ds(r128, 128)],
                    gbuf.at[parity, l], sems[parity]).start()

        def drain(parity):
            for l in range(wv):
                pltpu.make_async_copy(
                    t3_hbm.at[:, :, pl.ds(0, 128)],
                    gbuf.at[parity, l], sems[parity]).wait()

        def extract(w, parity):
            for l in range(wv):
                i = w * wv + l
                r = ism[i]
                lane = jnp.full((16,), r & 127, jnp.int32)
                thi = jnp.full((16,), lax.div(i, 128), jnp.int32)
                tlo = jnp.full((16,), lax.rem(i, 128), jnp.int32)
                pv = jnp.full((16,), parity, jnp.int32)
                lv = jnp.full((16,), l, jnp.int32)
                for j in range(d // 16):
                    vals = plsc.load_gather(
                        gbuf, [pv, lv, chunk_hi[j], chunk_lo[j], lane])
                    pev = plsc.load_gather(pe_v, [j * 16 + iota, thi, tlo])
                    plsc.store_scatter(
                        cols_v, [j * 16 + iota, thi, tlo], vals + pev)

        issue(0, 0)
        pe_cp.wait()

        def pipeline(k, carry):
            w0 = k * 2
            issue(w0 + 1, 1)
            drain(0)
            extract(w0, 0)

            @pl.when(w0 + 2 < nwaves)
            def _():
                issue(w0 + 2, 0)

            drain(1)
            extract(w0 + 1, 1)
            return carry

        lax.fori_loop(0, nwaves // 2, pipeline, 0)

        pltpu.sync_copy(cols_v, out_hbm.at[b, :, pl.ds(tg0, nt), :])

    out4 = emb(x_flat, pe_t, t3)
    return out4.reshape(batch, d, context).transpose(0, 2, 1)


# speedup vs baseline: 4.5033x; 1.0116x over previous
import functools

import jax
import jax.numpy as jnp
import numpy as np
from jax import lax
from jax.experimental import pallas as pl
from jax.experimental.pallas import tpu as pltpu
from jax.experimental.pallas import tpu_sc as plsc


def _positional_encoding(context_size, embedding_size):
    positions = np.arange(context_size, dtype=np.float32)
    indices = np.arange(embedding_size // 2, dtype=np.float32)
    scaling_factor = 10000.0 ** (2.0 * indices / embedding_size)
    angles = positions[:, None] / scaling_factor[None, :]
    pe = np.stack([np.sin(angles), np.cos(angles)], axis=-1)
    return jnp.asarray(pe.reshape(context_size, embedding_size))


def kernel(x, table):
    batch, context = x.shape
    v, d = table.shape
    n = batch * context

    info = plsc.get_sparse_core_info()
    nw = info.num_cores * info.num_subcores
    bpw = n // nw
    wv = 4
    nwaves = bpw // wv
    nt = bpw // 128

    pe_t = _positional_encoding(context, d).T.reshape(d, context // 128, 128)
    t3 = table.T.reshape(d // 8, 8, v)
    x_flat = x.reshape(n).astype(jnp.int32)

    mesh = plsc.VectorSubcoreMesh(core_axis_name="c", subcore_axis_name="s")

    @functools.partial(
        pl.kernel,
        out_type=jax.ShapeDtypeStruct((batch, d, context // 128, 128), jnp.float32),
        mesh=mesh,
        scratch_types=[
            pltpu.VMEM((bpw,), jnp.int32),
            pltpu.SMEM((bpw,), jnp.int32),
            pltpu.VMEM((2, wv, d // 8, 8, 128), jnp.float32),
            pltpu.VMEM((d, nt, 128), jnp.float32),
            pltpu.VMEM((d, nt, 128), jnp.float32),
            pltpu.SemaphoreType.DMA,
            pltpu.SemaphoreType.DMA,
            pltpu.SemaphoreType.DMA,
        ],
        compiler_params=pltpu.CompilerParams(needs_layout_passes=False),
    )
    def emb(x_hbm, pe_hbm, t3_hbm, out_hbm,
            idx_v, ism, gbuf, cols_v, pe_v, sem_a, sem_b, psem):
        wid = lax.axis_index("s") * info.num_cores + lax.axis_index("c")
        base = wid * bpw
        b = lax.div(base, context)
        t0 = pl.multiple_of(lax.rem(base, context), bpw)
        tg0 = lax.div(t0, 128)
        pe_cp = pltpu.make_async_copy(
            pe_hbm.at[:, pl.ds(tg0, nt), :], pe_v, psem)
        pe_cp.start()
        pltpu.sync_copy(x_hbm.at[pl.ds(base, bpw)], idx_v)

        def stage(q, carry):
            iv = idx_v[pl.ds(q * 16, 16)]
            for l in range(16):
                ism[q * 16 + l] = iv[l]
            return carry

        lax.fori_loop(0, bpw // 16, stage, 0)

        sems = [sem_a, sem_b]
        iota = lax.iota(jnp.int32, 16)
        chunk_hi = [(j * 16 + iota) >> 3 for j in range(d // 16)]
        chunk_lo = [(j * 16 + iota) & 7 for j in range(d // 16)]

        def issue(w, parity):
            for l in range(wv):
                r = ism[w * wv + l]
                r128 = pl.multiple_of((r >> 7) * 128, 128)
                for h in range(2):
                    pltpu.make_async_copy(
                        t3_hbm.at[pl.ds(h * 4, 4), :, pl.ds(r128, 128)],
                        gbuf.at[parity, l, pl.ds(h * 4, 4)],
                        sems[parity]).start()

        def drain(parity):
            for l in range(wv):
                for h in range(2):
                    pltpu.make_async_copy(
                        t3_hbm.at[pl.ds(h * 4, 4), :, pl.ds(0, 128)],
                        gbuf.at[parity, l, pl.ds(h * 4, 4)],
                        sems[parity]).wait()

        def extract(w, parity):
            for l in range(wv):
                i = w * wv + l
                r = ism[i]
                lane = jnp.full((16,), r & 127, jnp.int32)
                thi = jnp.full((16,), lax.div(i, 128), jnp.int32)
                tlo = jnp.full((16,), lax.rem(i, 128), jnp.int32)
                pv = jnp.full((16,), parity, jnp.int32)
                lv = jnp.full((16,), l, jnp.int32)
                for j in range(d // 16):
                    vals = plsc.load_gather(
                        gbuf, [pv, lv, chunk_hi[j], chunk_lo[j], lane])
                    pev = plsc.load_gather(pe_v, [j * 16 + iota, thi, tlo])
                    plsc.store_scatter(
                        cols_v, [j * 16 + iota, thi, tlo], vals + pev)

        issue(0, 0)
        pe_cp.wait()

        def pipeline(k, carry):
            w0 = k * 2
            issue(w0 + 1, 1)
            drain(0)
            extract(w0, 0)

            @pl.when(w0 + 2 < nwaves)
            def _():
                issue(w0 + 2, 0)

            drain(1)
            extract(w0 + 1, 1)
            return carry

        lax.fori_loop(0, nwaves // 2, pipeline, 0)

        pltpu.sync_copy(cols_v, out_hbm.at[b, :, pl.ds(tg0, nt), :])

    out4 = emb(x_flat, pe_t, t3)
    return out4.reshape(batch, d, context).transpose(0, 2, 1)


# speedup vs baseline: 4.8177x; 1.0698x over previous
import functools

import jax
import jax.numpy as jnp
import numpy as np
from jax import lax
from jax.experimental import pallas as pl
from jax.experimental.pallas import tpu as pltpu
from jax.experimental.pallas import tpu_sc as plsc


def _positional_encoding(context_size, embedding_size):
    positions = np.arange(context_size, dtype=np.float32)
    indices = np.arange(embedding_size // 2, dtype=np.float32)
    scaling_factor = 10000.0 ** (2.0 * indices / embedding_size)
    angles = positions[:, None] / scaling_factor[None, :]
    pe = np.stack([np.sin(angles), np.cos(angles)], axis=-1)
    return jnp.asarray(pe.reshape(context_size, embedding_size))


def kernel(x, table):
    batch, context = x.shape
    v, d = table.shape
    n = batch * context

    info = plsc.get_sparse_core_info()
    nw = info.num_cores * info.num_subcores
    bpw = n // nw
    wv = 4
    nbuf = 3
    nwaves = bpw // wv
    nt = bpw // 128

    pe_t = _positional_encoding(context, d).T.reshape(d, context // 128, 128)
    t3 = table.T.reshape(d // 8, 8, v)
    x_flat = x.reshape(n).astype(jnp.int32)

    mesh = plsc.VectorSubcoreMesh(core_axis_name="c", subcore_axis_name="s")

    @functools.partial(
        pl.kernel,
        out_type=jax.ShapeDtypeStruct((batch, d, context // 128, 128), jnp.float32),
        mesh=mesh,
        scratch_types=[
            pltpu.VMEM((bpw,), jnp.int32),
            pltpu.SMEM((bpw,), jnp.int32),
            pltpu.VMEM((nbuf, wv, d // 8, 8, 128), jnp.float32),
            pltpu.VMEM((d, nt, 128), jnp.float32),
            pltpu.SemaphoreType.DMA,
            pltpu.SemaphoreType.DMA,
            pltpu.SemaphoreType.DMA,
            pltpu.SemaphoreType.DMA,
        ],
        compiler_params=pltpu.CompilerParams(needs_layout_passes=False),
    )
    def emb(x_hbm, pe_hbm, t3_hbm, out_hbm,
            idx_v, ism, gbuf, cols_v, sem_a, sem_b, sem_c, psem):
        wid = lax.axis_index("s") * info.num_cores + lax.axis_index("c")
        base = wid * bpw
        b = lax.div(base, context)
        t0 = pl.multiple_of(lax.rem(base, context), bpw)
        tg0 = lax.div(t0, 128)
        pe_cp = pltpu.make_async_copy(
            pe_hbm.at[:, pl.ds(tg0, nt), :], cols_v, psem)
        pe_cp.start()
        pltpu.sync_copy(x_hbm.at[pl.ds(base, bpw)], idx_v)

        def stage(q, carry):
            iv = idx_v[pl.ds(q * 16, 16)]
            for l in range(16):
                ism[q * 16 + l] = iv[l]
            return carry

        lax.fori_loop(0, bpw // 16, stage, 0)

        sems = [sem_a, sem_b, sem_c]
        iota = lax.iota(jnp.int32, 16)
        chunk_hi = [(j * 16 + iota) >> 3 for j in range(d // 16)]
        chunk_lo = [(j * 16 + iota) & 7 for j in range(d // 16)]

        def issue(w, parity):
            for l in range(wv):
                r = ism[w * wv + l]
                r128 = pl.multiple_of((r >> 7) * 128, 128)
                for h in range(2):
                    pltpu.make_async_copy(
                        t3_hbm.at[pl.ds(h * 4, 4), :, pl.ds(r128, 128)],
                        gbuf.at[parity, l, pl.ds(h * 4, 4)],
                        sems[parity]).start()

        def drain(parity):
            for l in range(wv):
                for h in range(2):
                    pltpu.make_async_copy(
                        t3_hbm.at[pl.ds(h * 4, 4), :, pl.ds(0, 128)],
                        gbuf.at[parity, l, pl.ds(h * 4, 4)],
                        sems[parity]).wait()

        def extract(w, parity):
            for l in range(wv):
                i = w * wv + l
                r = ism[i]
                lane = jnp.full((16,), r & 127, jnp.int32)
                thi = jnp.full((16,), lax.div(i, 128), jnp.int32)
                tlo = jnp.full((16,), lax.rem(i, 128), jnp.int32)
                pv = jnp.full((16,), parity, jnp.int32)
                lv = jnp.full((16,), l, jnp.int32)
                for j in range(d // 16):
                    vals = plsc.load_gather(
                        gbuf, [pv, lv, chunk_hi[j], chunk_lo[j], lane])
                    pev = plsc.load_gather(cols_v, [j * 16 + iota, thi, tlo])
                    plsc.store_scatter(
                        cols_v, [j * 16 + iota, thi, tlo], vals + pev)

        issue(0, 0)
        issue(1, 1)
        pe_cp.wait()

        def pipeline(k, carry):
            w0 = k * nbuf
            for s in range(nbuf):
                issue(w0 + s + 2, (s + 2) % nbuf)
                drain(s)
                extract(w0 + s, s)
            return carry

        lax.fori_loop(0, nwaves // nbuf - 1, pipeline, 0)
        done = (nwaves // nbuf - 1) * nbuf
        for w in range(done, nwaves):
            if w + 2 < nwaves:
                issue(w + 2, (w + 2) % nbuf)
            drain(w % nbuf)
            extract(w, w % nbuf)

        pltpu.sync_copy(cols_v, out_hbm.at[b, :, pl.ds(tg0, nt), :])

    out4 = emb(x_flat, pe_t, t3)
    return out4.reshape(batch, d, context).transpose(0, 2, 1)


# speedup vs baseline: 5.2901x; 1.0980x over previous
import functools

import jax
import jax.numpy as jnp
import numpy as np
from jax import lax
from jax.experimental import pallas as pl
from jax.experimental.pallas import tpu as pltpu
from jax.experimental.pallas import tpu_sc as plsc


def _positional_encoding(context_size, embedding_size):
    positions = np.arange(context_size, dtype=np.float32)
    indices = np.arange(embedding_size // 2, dtype=np.float32)
    scaling_factor = 10000.0 ** (2.0 * indices / embedding_size)
    angles = positions[:, None] / scaling_factor[None, :]
    pe = np.stack([np.sin(angles), np.cos(angles)], axis=-1)
    return jnp.asarray(pe.reshape(context_size, embedding_size))


def kernel(x, table):
    batch, context = x.shape
    v, d = table.shape
    n = batch * context

    info = plsc.get_sparse_core_info()
    nw = info.num_cores * info.num_subcores
    bpw = n // nw
    wv = 2
    nbuf = 6
    nwaves = bpw // wv
    nt = bpw // 128

    pe_t = _positional_encoding(context, d).T.reshape(d, context // 128, 128)
    t3 = table.T.reshape(d // 8, 8, v)
    x_flat = x.reshape(n).astype(jnp.int32)

    mesh = plsc.VectorSubcoreMesh(core_axis_name="c", subcore_axis_name="s")

    @functools.partial(
        pl.kernel,
        out_type=jax.ShapeDtypeStruct((batch, d, context // 128, 128), jnp.float32),
        mesh=mesh,
        scratch_types=[
            pltpu.VMEM((bpw,), jnp.int32),
            pltpu.SMEM((bpw,), jnp.int32),
            pltpu.VMEM((nbuf, wv, d // 8, 8, 128), jnp.float32),
            pltpu.VMEM((d, nt, 128), jnp.float32),
            [pltpu.SemaphoreType.DMA] * nbuf,
            pltpu.SemaphoreType.DMA,
        ],
        compiler_params=pltpu.CompilerParams(needs_layout_passes=False),
    )
    def emb(x_hbm, pe_hbm, t3_hbm, out_hbm,
            idx_v, ism, gbuf, cols_v, sems, psem):
        wid = lax.axis_index("s") * info.num_cores + lax.axis_index("c")
        base = wid * bpw
        b = lax.div(base, context)
        t0 = pl.multiple_of(lax.rem(base, context), bpw)
        tg0 = lax.div(t0, 128)
        pe_cp = pltpu.make_async_copy(
            pe_hbm.at[:, pl.ds(tg0, nt), :], cols_v, psem)
        pe_cp.start()
        pltpu.sync_copy(x_hbm.at[pl.ds(base, bpw)], idx_v)

        def stage(q, carry):
            iv = idx_v[pl.ds(q * 16, 16)]
            for l in range(16):
                ism[q * 16 + l] = iv[l]
            return carry

        lax.fori_loop(0, bpw // 16, stage, 0)

        iota = lax.iota(jnp.int32, 16)
        chunk_hi = [(j * 16 + iota) >> 3 for j in range(d // 16)]
        chunk_lo = [(j * 16 + iota) & 7 for j in range(d // 16)]

        def issue(w, parity):
            for l in range(wv):
                r = ism[w * wv + l]
                r128 = pl.multiple_of((r >> 7) * 128, 128)
                for h in range(2):
                    pltpu.make_async_copy(
                        t3_hbm.at[pl.ds(h * 4, 4), :, pl.ds(r128, 128)],
                        gbuf.at[parity, l, pl.ds(h * 4, 4)],
                        sems[parity]).start()

        def drain(parity):
            for l in range(wv):
                for h in range(2):
                    pltpu.make_async_copy(
                        t3_hbm.at[pl.ds(h * 4, 4), :, pl.ds(0, 128)],
                        gbuf.at[parity, l, pl.ds(h * 4, 4)],
                        sems[parity]).wait()

        def extract(w, parity):
            for l in range(wv):
                i = w * wv + l
                r = ism[i]
                lane = jnp.full((16,), r & 127, jnp.int32)
                thi = jnp.full((16,), lax.div(i, 128), jnp.int32)
                tlo = jnp.full((16,), lax.rem(i, 128), jnp.int32)
                pv = jnp.full((16,), parity, jnp.int32)
                lv = jnp.full((16,), l, jnp.int32)
                for j in range(d // 16):
                    vals = plsc.load_gather(
                        gbuf, [pv, lv, chunk_hi[j], chunk_lo[j], lane])
                    pev = plsc.load_gather(cols_v, [j * 16 + iota, thi, tlo])
                    plsc.store_scatter(
                        cols_v, [j * 16 + iota, thi, tlo], vals + pev)

        ahead = nbuf - 1
        for w in range(ahead):
            issue(w, w % nbuf)
        pe_cp.wait()

        def pipeline(k, carry):
            w0 = k * nbuf
            for s in range(nbuf):
                issue(w0 + s + ahead, (s + ahead) % nbuf)
                drain(s)
                extract(w0 + s, s)
            return carry

        nloop = (nwaves - ahead) // nbuf
        lax.fori_loop(0, nloop, pipeline, 0)
        for w in range(nloop * nbuf, nwaves):
            if w + ahead < nwaves:
                issue(w + ahead, (w + ahead) % nbuf)
            drain(w % nbuf)
            extract(w, w % nbuf)

        pltpu.sync_copy(cols_v, out_hbm.at[b, :, pl.ds(tg0, nt), :])

    out4 = emb(x_flat, pe_t, t3)
    return out4.reshape(batch, d, context).transpose(0, 2, 1)
